# baseline (speedup 1.0000x reference)
import functools

import jax
import jax.numpy as jnp
from jax.experimental import pallas as pl
from jax.experimental.pallas import tpu as pltpu

_INTERPRET = False
_H = 8


def _ln(x, g, b, eps=1e-5):
    m = jnp.mean(x, -1, keepdims=True)
    v = jnp.mean((x - m) ** 2, -1, keepdims=True)
    return (x - m) * jax.lax.rsqrt(v + eps) * g + b


def _pick_tile(total, want):
    t = min(want, total)
    while total % t:
        t //= 2
    return t


def _cparams(sem, vmem_mb=50):
    return pltpu.CompilerParams(
        dimension_semantics=sem, vmem_limit_bytes=vmem_mb * 1024 * 1024)


def _kv_body(x_ref, g_ref, b_ref, wk_ref, wv_ref, k_ref, v_ref, *, li):
    xn = _ln(x_ref[...], g_ref[li:li + 1, :], b_ref[li:li + 1, :])
    k_ref[...] = jnp.dot(xn, wk_ref[0], preferred_element_type=jnp.float32)
    v_ref[...] = jnp.dot(xn, wv_ref[0], preferred_element_type=jnp.float32)


def _kv(xf, li, g, b, wk, wv):
    BN, D = xf.shape
    TN = _pick_tile(BN, 256)
    grid = (BN // TN,)
    L = g.shape[0]
    row = pl.BlockSpec((TN, D), lambda i: (i, 0))
    full = pl.BlockSpec((1, D, D), lambda i: (li, 0, 0))
    vec = pl.BlockSpec((L, D), lambda i: (0, 0))
    out = jax.ShapeDtypeStruct((BN, D), jnp.float32)
    return pl.pallas_call(
        functools.partial(_kv_body, li=li),
        grid=grid,
        in_specs=[row, vec, vec, full, full],
        out_specs=[row, row],
        out_shape=[out, out],
        compiler_params=_cparams(("parallel",)),
        name="ln_kv",
        interpret=_INTERPRET,
    )(xf, g, b, wk, wv)


def _lnkvproj_body(x_ref, g_ref, b_ref, wk_ref, wv_ref, pk_ref, pv_ref,
                   ko_ref, vo_ref, *, li):
    j = pl.program_id(1)
    xn = _ln(x_ref[0], g_ref[li:li + 1, :], b_ref[li:li + 1, :])
    ke = jnp.dot(xn, wk_ref[0], preferred_element_type=jnp.float32)
    va = jnp.dot(xn, wv_ref[0], preferred_element_type=jnp.float32)
    kk = jax.lax.dot_general(pk_ref[0], ke, (((0,), (0,)), ((), ())),
                             preferred_element_type=jnp.float32)
    vv = jax.lax.dot_general(pv_ref[0], va, (((0,), (0,)), ((), ())),
                             preferred_element_type=jnp.float32)

    @pl.when(j == 0)
    def _():
        ko_ref[0] = kk
        vo_ref[0] = vv

    @pl.when(j != 0)
    def _():
        ko_ref[0] += kk
        vo_ref[0] += vv


def _lnkvproj(x3, li, g, b, wk, wv, pk, pv):
    B, N, D = x3.shape
    K = pk.shape[2]
    L = g.shape[0]
    TN = _pick_tile(N, 512)
    grid = (B, N // TN)
    row = pl.BlockSpec((1, TN, D), lambda bb, j: (bb, j, 0))
    full = pl.BlockSpec((1, D, D), lambda bb, j: (li, 0, 0))
    vec = pl.BlockSpec((L, D), lambda bb, j: (0, 0))
    proj = pl.BlockSpec((1, TN, K), lambda bb, j: (li, j, 0))
    out = pl.BlockSpec((1, K, D), lambda bb, j: (bb, 0, 0))
    osh = jax.ShapeDtypeStruct((B, K, D), jnp.float32)
    return pl.pallas_call(
        functools.partial(_lnkvproj_body, li=li),
        grid=grid,
        in_specs=[row, vec, vec, full, full, proj, proj],
        out_specs=[out, out],
        out_shape=[osh, osh],
        compiler_params=_cparams(("parallel", "arbitrary")),
        name="ln_kv_proj",
        interpret=_INTERPRET,
    )(x3, g, b, wk, wv, pk, pv)


def _convproj_body(wk_hbm, wv_hbm, ke_hbm, va_hbm, ko_ref, vo_ref,
                   wkb, wvb, keb, vab, sem, *, li):
    S = wk_hbm.shape[2]
    Bn = ke_hbm.shape[0]
    s = pl.program_id(0)

    def start(sidx, slot):
        pltpu.make_async_copy(wk_hbm.at[li, :, pl.ds(sidx, 1), :], wkb.at[slot],
                              sem.at[slot, 0]).start()
        pltpu.make_async_copy(wv_hbm.at[li, :, pl.ds(sidx, 1), :], wvb.at[slot],
                              sem.at[slot, 1]).start()
        pltpu.make_async_copy(ke_hbm.at[:, :, pl.ds(sidx, 1), :], keb.at[slot],
                              sem.at[slot, 2]).start()
        pltpu.make_async_copy(va_hbm.at[:, :, pl.ds(sidx, 1), :], vab.at[slot],
                              sem.at[slot, 3]).start()

    slot = jax.lax.rem(s, 2)

    @pl.when(s == 0)
    def _():
        start(0, 0)

    @pl.when(s < S - 1)
    def _():
        start(s + 1, 1 - slot)

    pltpu.make_async_copy(wkb.at[slot], wkb.at[slot], sem.at[slot, 0]).wait()
    pltpu.make_async_copy(wvb.at[slot], wvb.at[slot], sem.at[slot, 1]).wait()
    pltpu.make_async_copy(keb.at[slot], keb.at[slot], sem.at[slot, 2]).wait()
    pltpu.make_async_copy(vab.at[slot], vab.at[slot], sem.at[slot, 3]).wait()

    dn = (((1,), (1,)), ((), ()))
    wk = wkb[slot, :, 0, :]
    wv = wvb[slot, :, 0, :]
    for bb in range(Bn):
        kk = jax.lax.dot_general(keb[slot, bb, :, 0, :], wk, dn,
                                 preferred_element_type=jnp.float32)
        vv = jax.lax.dot_general(vab[slot, bb, :, 0, :], wv, dn,
                                 preferred_element_type=jnp.float32)

        @pl.when(s == 0)
        def _(bb=bb, kk=kk, vv=vv):
            ko_ref[bb] = kk
            vo_ref[bb] = vv

        @pl.when(s != 0)
        def _(bb=bb, kk=kk, vv=vv):
            ko_ref[bb] += kk
            vo_ref[bb] += vv


def _convproj(ke4, va4, li, wkp, wvp):
    B, K, S, D = ke4.shape
    O = wkp.shape[1]
    osh = jax.ShapeDtypeStruct((B, K, D), jnp.float32)
    out = pl.BlockSpec((B, K, D), lambda s: (0, 0, 0))
    anyspec = pl.BlockSpec(memory_space=pl.ANY)
    return pl.pallas_call(
        functools.partial(_convproj_body, li=li),
        grid=(S,),
        in_specs=[anyspec, anyspec, anyspec, anyspec],
        out_specs=[out, out],
        out_shape=[osh, osh],
        scratch_shapes=[
            pltpu.VMEM((2, O, 1, D), jnp.float32),
            pltpu.VMEM((2, O, 1, D), jnp.float32),
            pltpu.VMEM((2, B, K, 1, D), jnp.float32),
            pltpu.VMEM((2, B, K, 1, D), jnp.float32),
            pltpu.SemaphoreType.DMA((2, 4)),
        ],
        compiler_params=_cparams(("arbitrary",)),
        name="conv_kv_proj",
        interpret=_INTERPRET,
    )(wkp, wvp, ke4, va4)


def _attn_ffn_body(x_ref, g1_ref, b1n_ref, wq_ref, k_ref, v_ref, wo_ref, bo_ref,
                   g_ref, b_ref, w1_ref, b1_ref, w2_ref, b2_ref, o_ref, *, nchunk, li):
    xn1 = _ln(x_ref[0], g1_ref[li:li + 1, :], b1n_ref[li:li + 1, :])
    q = jnp.dot(xn1, wq_ref[0], preferred_element_type=jnp.float32)
    k = k_ref[0]
    v = v_ref[0]
    D = q.shape[1]
    DH = D // _H
    scale = DH ** -0.5
    outs = []
    for h in range(_H):
        sl = slice(h * DH, (h + 1) * DH)
        dots = jax.lax.dot_general(q[:, sl], k[:, sl], (((1,), (1,)), ((), ())),
                                   preferred_element_type=jnp.float32) * scale
        m = jnp.max(dots, axis=-1, keepdims=True)
        p = jnp.exp(dots - m)
        l = jnp.sum(p, axis=-1, keepdims=True)
        a = p / l
        outs.append(jax.lax.dot_general(a, v[:, sl], (((1,), (0,)), ((), ())),
                                        preferred_element_type=jnp.float32))
    o = jnp.concatenate(outs, axis=-1)
    x1 = x_ref[0] + jnp.dot(o, wo_ref[0], preferred_element_type=jnp.float32) + bo_ref[li:li + 1, :]
    xn = _ln(x1, g_ref[li:li + 1, :], b_ref[li:li + 1, :])
    DFF = w1_ref.shape[2]
    CF = DFF // nchunk
    o_ref[0] = x1 + b2_ref[li:li + 1, :]
    for c in range(nchunk):
        slc = slice(c * CF, (c + 1) * CF)
        h = jnp.dot(xn, w1_ref[0, :, slc], preferred_element_type=jnp.float32) + b1_ref[li:li + 1, slc]
        h = 0.5 * h * (1.0 + jax.lax.erf(h * (2.0 ** -0.5)))
        o_ref[0] += jnp.dot(h, w2_ref[0, slc, :], preferred_element_type=jnp.float32)


def _attn_ffn(x3, li, g1, b1n, wq, k_, v_, wo, bo, g, b, w1, b1, w2, b2):
    B, N, D = x3.shape
    K = k_.shape[1]
    DFF = w1.shape[2]
    TQ = _pick_tile(N, 256)
    CF = _pick_tile(DFF, 512)
    L = bo.shape[0]
    grid = (B, N // TQ)
    row = pl.BlockSpec((1, TQ, D), lambda bb, n: (bb, n, 0))
    kv = pl.BlockSpec((1, K, D), lambda bb, n: (bb, 0, 0))
    full = pl.BlockSpec((1, D, D), lambda bb, n: (li, 0, 0))
    vec = pl.BlockSpec((L, D), lambda bb, n: (0, 0))
    return pl.pallas_call(
        functools.partial(_attn_ffn_body, nchunk=DFF // CF, li=li),
        grid=grid,
        in_specs=[row, vec, vec, full, kv, kv, full, vec,
                  vec, vec,
                  pl.BlockSpec((1, D, DFF), lambda bb, n: (li, 0, 0)),
                  pl.BlockSpec((L, DFF), lambda bb, n: (0, 0)),
                  pl.BlockSpec((1, DFF, D), lambda bb, n: (li, 0, 0)),
                  vec],
        out_specs=row,
        out_shape=jax.ShapeDtypeStruct((B, N, D), jnp.float32),
        compiler_params=_cparams(("parallel", "parallel"), vmem_mb=56),
        name="attn_ffn",
        interpret=_INTERPRET,
    )(x3, g1, b1n, wq, k_, v_, wo, bo, g, b, w1, b1, w2, b2)


def kernel(x, lin_ln1_g, lin_ln1_b, lin_wq, lin_wk, lin_wv, lin_pk, lin_pv,
           lin_wo, lin_bo, lin_ln2_g, lin_ln2_b, lin_w1, lin_b1, lin_w2, lin_b2,
           conv_ln1_g, conv_ln1_b, conv_wq, conv_wk, conv_wv, conv_pk, conv_pv,
           conv_wo, conv_bo, conv_ln2_g, conv_ln2_b, conv_w1, conv_b1, conv_w2, conv_b2):
    B, N, D = x.shape
    L = lin_wq.shape[0]
    K = lin_pk.shape[2]
    S = conv_pk.shape[3]
    xf = x.reshape(B * N, D)

    x3 = x
    for i in range(L):
        k_, v_ = _lnkvproj(x3, i, lin_ln1_g, lin_ln1_b, lin_wk, lin_wv,
                           lin_pk, lin_pv)
        x3 = _attn_ffn(x3, i, lin_ln1_g, lin_ln1_b, lin_wq, k_, v_,
                       lin_wo, lin_bo, lin_ln2_g, lin_ln2_b,
                       lin_w1, lin_b1, lin_w2, lin_b2)

    pkT_all = jnp.transpose(conv_pk, (0, 1, 3, 2))
    pvT_all = jnp.transpose(conv_pv, (0, 1, 3, 2))
    for i in range(L):
        xf = x3.reshape(B * N, D)
        ke, va = _kv(xf, i, conv_ln1_g, conv_ln1_b, conv_wk, conv_wv)
        k_, v_ = _convproj(ke.reshape(B, K, S, D), va.reshape(B, K, S, D), i,
                           pkT_all, pvT_all)
        x3 = _attn_ffn(x3, i, conv_ln1_g, conv_ln1_b, conv_wq, k_, v_,
                       conv_wo, conv_bo, conv_ln2_g, conv_ln2_b,
                       conv_w1, conv_b1, conv_w2, conv_b2)

    return x3
